# baseline (speedup 1.0000x reference)
import math

import jax
import jax.numpy as jnp
from jax import lax
from jax.experimental import pallas as pl
from jax.experimental.pallas import tpu as pltpu
from jax.experimental.pallas import tpu_sc as plsc

_LANES = 16


def _positional_encoding_2d(seq_len, d):
    pos = jnp.arange(seq_len, dtype=jnp.float32)[:, None]
    even_idx = jnp.arange(0, d, 2, dtype=jnp.float32)
    odd_idx = jnp.arange(1, d, 2, dtype=jnp.float32)
    even_div = jnp.power(10000.0, 2.0 * even_idx / d)
    odd_div = jnp.power(10000.0, 2.0 * odd_idx / d)
    pe = jnp.zeros((seq_len, d), dtype=jnp.float32)
    pe = pe.at[:, 0::2].set(jnp.sin(pos / even_div))
    pe = pe.at[:, 1::2].set(jnp.cos(pos / odd_div))
    return pe


def kernel(x, table):
    b, s = x.shape
    v, d = table.shape
    scale = math.sqrt(d)

    info = plsc.get_sparse_core_info()
    nc, ns = info.num_cores, info.num_subcores

    sub_tok = 128
    nsub = 2
    stripe = nsub * sub_tok
    sper = s // nc
    assert b == ns * stripe and s % nc == 0 and d % _LANES == 0
    assert sper % 2 == 0
    groups = d // _LANES

    pe_flat = _positional_encoding_2d(s, d).reshape(-1)
    xtr = x.astype(jnp.int32).T.reshape(s, ns, nsub, sub_tok)

    mesh = plsc.VectorSubcoreMesh(core_axis_name="c", subcore_axis_name="s")

    def body(x_hbm, pe_hbm, table_hbm, out_hbm,
             idx_v, pe_v, gbuf0, gbuf1, wbig0, wbig1,
             gsem0, gsem1, wsem0, wsem1):
        g = lax.axis_index("c")
        l = lax.axis_index("s")
        lane = lax.iota(jnp.int32, _LANES)
        lanefb = [lane + fb * _LANES for fb in range(groups)]
        s0 = g * sper
        pltpu.sync_copy(x_hbm.at[pl.ds(s0, sper), l], idx_v)
        pltpu.sync_copy(pe_hbm.at[pl.ds(s0 * d, sper * d)], pe_v)

        def issue_gather(si, sub, gbuf, gsem):
            pltpu.async_copy(table_hbm.at[idx_v.at[si, sub]], gbuf, gsem)

        def wait_gather(si, sub, gbuf, gsem):
            pltpu.make_async_copy(
                table_hbm.at[idx_v.at[si, sub]], gbuf, gsem).wait()

        def stripe_dma(si, wbig, wsem):
            return pltpu.make_async_copy(
                wbig, out_hbm.at[s0 + si, :, pl.ds(l * stripe, stripe)], wsem)

        issue_gather(0, 0, gbuf0, gsem0)
        issue_gather(0, 1, gbuf1, gsem1)

        @pl.loop(0, sper // 2)
        def _outer(t):
            for pos_par, sub, gbuf, gsem, wbig, wsem in (
                (0, 0, gbuf0, gsem0, wbig0, wsem0),
                (0, 1, gbuf1, gsem1, wbig0, wsem0),
                (1, 0, gbuf0, gsem0, wbig1, wsem1),
                (1, 1, gbuf1, gsem1, wbig1, wsem1),
            ):
                si = 2 * t + pos_par
                wait_gather(si, sub, gbuf, gsem)

                if sub == 0:
                    @pl.when(t > 0)
                    def _drain_prev_write():
                        stripe_dma(si - 2, wbig, wsem).wait()

                peg = [pe_v[pl.ds(si * d + fb * _LANES, _LANES)]
                       for fb in range(groups)]

                @pl.loop(0, sub_tok)
                def _diag(i):
                    rot = (lane + i) & 15
                    tb = i & ~15
                    rowr = rot + tb
                    colw = rot + (sub * sub_tok + tb)
                    for fb in range(groups):
                        v = plsc.load_gather(gbuf, [rowr, lanefb[fb]])
                        plsc.store_scatter(wbig, [lanefb[fb], colw],
                                           v * scale + peg[fb])

                if sub == nsub - 1:
                    stripe_dma(si, wbig, wsem).start()

                nxt = 2 * si + sub + 2
                @pl.when(nxt < 2 * sper)
                def _next_gather():
                    issue_gather(lax.div(nxt, 2), lax.rem(nxt, 2), gbuf, gsem)

        stripe_dma(sper - 2, wbig0, wsem0).wait()
        stripe_dma(sper - 1, wbig1, wsem1).wait()

    out_phys = pl.kernel(
        body,
        out_type=jax.ShapeDtypeStruct((s, d, b), jnp.float32),
        mesh=mesh,
        compiler_params=pltpu.CompilerParams(use_tc_tiling_on_sc=False,
                                             needs_layout_passes=False),
        scratch_types=[
            pltpu.VMEM((sper, nsub, sub_tok), jnp.int32),
            pltpu.VMEM((sper * d,), jnp.float32),
            pltpu.VMEM((sub_tok, d), jnp.float32),
            pltpu.VMEM((sub_tok, d), jnp.float32),
            pltpu.VMEM((d, stripe), jnp.float32),
            pltpu.VMEM((d, stripe), jnp.float32),
            pltpu.SemaphoreType.DMA,
            pltpu.SemaphoreType.DMA,
            pltpu.SemaphoreType.DMA,
            pltpu.SemaphoreType.DMA,
        ],
    )(xtr, pe_flat, table)
    return out_phys.transpose(2, 0, 1)


# speedup vs baseline: 1.1958x; 1.1958x over previous
import math

import jax
import jax.numpy as jnp
from jax import lax
from jax.experimental import pallas as pl
from jax.experimental.pallas import tpu as pltpu
from jax.experimental.pallas import tpu_sc as plsc

_LANES = 16


def _positional_encoding_2d(seq_len, d):
    pos = jnp.arange(seq_len, dtype=jnp.float32)[:, None]
    even_idx = jnp.arange(0, d, 2, dtype=jnp.float32)
    odd_idx = jnp.arange(1, d, 2, dtype=jnp.float32)
    even_div = jnp.power(10000.0, 2.0 * even_idx / d)
    odd_div = jnp.power(10000.0, 2.0 * odd_idx / d)
    pe = jnp.zeros((seq_len, d), dtype=jnp.float32)
    pe = pe.at[:, 0::2].set(jnp.sin(pos / even_div))
    pe = pe.at[:, 1::2].set(jnp.cos(pos / odd_div))
    return pe


def kernel(x, table):
    b, s = x.shape
    v, d = table.shape
    scale = math.sqrt(d)

    info = plsc.get_sparse_core_info()
    nc, ns = info.num_cores, info.num_subcores

    sub_tok = 128
    nsub = 2
    stripe = nsub * sub_tok
    sper = s // nc
    assert b == ns * stripe and s % nc == 0 and d % _LANES == 0
    nchunk = sper * nsub
    assert nchunk % 4 == 0
    groups = d // _LANES

    pe_flat = _positional_encoding_2d(s, d).reshape(-1)
    xtr = x.astype(jnp.int32).T.reshape(s, ns, nsub, sub_tok)

    mesh = plsc.VectorSubcoreMesh(core_axis_name="c", subcore_axis_name="s")

    def body(x_hbm, pe_hbm, table_hbm, out_hbm,
             idx_v, pe_v, buf0, buf1, buf2, buf3,
             gsem0, gsem1, gsem2, gsem3, wsem0, wsem1, wsem2, wsem3):
        g = lax.axis_index("c")
        l = lax.axis_index("s")
        s0 = g * sper
        pltpu.sync_copy(x_hbm.at[pl.ds(s0, sper), l], idx_v)
        pltpu.sync_copy(pe_hbm.at[pl.ds(s0 * d, sper * d)], pe_v)

        bufs = (buf0, buf1, buf2, buf3)
        gsems = (gsem0, gsem1, gsem2, gsem3)
        wsems = (wsem0, wsem1, wsem2, wsem3)

        def gather(m, bb, gsem):
            return pltpu.make_async_copy(
                table_hbm.at[idx_v.at[lax.div(m, nsub), lax.rem(m, nsub)]],
                bb, gsem)

        def write(m, bb, wsem):
            si = lax.div(m, nsub)
            sub = lax.rem(m, nsub)
            return pltpu.make_async_copy(
                bb,
                out_hbm.at[s0 + si,
                           pl.ds(l * stripe + sub * sub_tok, sub_tok), :],
                wsem)

        for p in range(2):
            gather(p, bufs[p], gsems[p]).start()

        @pl.loop(0, nchunk // 4)
        def _outer(t):
            for p in range(4):
                m = 4 * t + p
                si = lax.div(m, nsub)
                gather(m, bufs[p], gsems[p]).wait()

                peg = [pe_v[pl.ds(si * d + fb * _LANES, _LANES)]
                       for fb in range(groups)]

                @pl.loop(0, sub_tok, step=2)
                def _token(j0):
                    vecs = [bufs[p][j, pl.ds(fb * _LANES, _LANES)]
                            for j in (j0, j0 + 1) for fb in range(groups)]
                    scaled = [vv * scale for vv in vecs]
                    added = [scaled[tj * groups + fb] + peg[fb]
                             for tj in range(2) for fb in range(groups)]
                    for tj in (0, 1):
                        for fb in range(groups):
                            bufs[p][j0 + tj, pl.ds(fb * _LANES, _LANES)] = (
                                added[tj * groups + fb])

                write(m, bufs[p], wsems[p]).start()

                pn = (p + 2) % 4
                @pl.when(m + 2 < nchunk)
                def _next_gather():
                    @pl.when(m >= 2)
                    def _drain():
                        write(m - 2, bufs[pn], wsems[pn]).wait()
                    gather(m + 2, bufs[pn], gsems[pn]).start()

        for p in range(4):
            write(nchunk - 4 + p, bufs[p], wsems[p]).wait()

    out_phys = pl.kernel(
        body,
        out_type=jax.ShapeDtypeStruct((s, b, d), jnp.float32),
        mesh=mesh,
        compiler_params=pltpu.CompilerParams(use_tc_tiling_on_sc=False,
                                             needs_layout_passes=False),
        scratch_types=[
            pltpu.VMEM((sper, nsub, sub_tok), jnp.int32),
            pltpu.VMEM((sper * d,), jnp.float32),
            pltpu.VMEM((sub_tok, d), jnp.float32),
            pltpu.VMEM((sub_tok, d), jnp.float32),
            pltpu.VMEM((sub_tok, d), jnp.float32),
            pltpu.VMEM((sub_tok, d), jnp.float32),
            pltpu.SemaphoreType.DMA,
            pltpu.SemaphoreType.DMA,
            pltpu.SemaphoreType.DMA,
            pltpu.SemaphoreType.DMA,
            pltpu.SemaphoreType.DMA,
            pltpu.SemaphoreType.DMA,
            pltpu.SemaphoreType.DMA,
            pltpu.SemaphoreType.DMA,
        ],
    )(xtr, pe_flat, table)
    return out_phys.transpose(1, 0, 2)


# speedup vs baseline: 1.1985x; 1.0022x over previous
import math

import jax
import jax.numpy as jnp
from jax import lax
from jax.experimental import pallas as pl
from jax.experimental.pallas import tpu as pltpu
from jax.experimental.pallas import tpu_sc as plsc

_LANES = 16


def _positional_encoding_2d(seq_len, d):
    pos = jnp.arange(seq_len, dtype=jnp.float32)[:, None]
    even_idx = jnp.arange(0, d, 2, dtype=jnp.float32)
    odd_idx = jnp.arange(1, d, 2, dtype=jnp.float32)
    even_div = jnp.power(10000.0, 2.0 * even_idx / d)
    odd_div = jnp.power(10000.0, 2.0 * odd_idx / d)
    pe = jnp.zeros((seq_len, d), dtype=jnp.float32)
    pe = pe.at[:, 0::2].set(jnp.sin(pos / even_div))
    pe = pe.at[:, 1::2].set(jnp.cos(pos / odd_div))
    return pe


def kernel(x, table):
    b, s = x.shape
    v, d = table.shape
    scale = math.sqrt(d)

    info = plsc.get_sparse_core_info()
    nc, ns = info.num_cores, info.num_subcores

    sub_tok = 128
    nsub = 2
    stripe = nsub * sub_tok
    sper = s // nc
    assert b == ns * stripe and s % nc == 0 and d % _LANES == 0
    nchunk = sper * nsub
    assert nchunk % 4 == 0
    groups = d // _LANES

    pe_flat = _positional_encoding_2d(s, d).reshape(-1)
    xtr = x.astype(jnp.int32).T.reshape(s, ns, nsub, sub_tok)

    mesh = plsc.VectorSubcoreMesh(core_axis_name="c", subcore_axis_name="s")

    def body(x_hbm, pe_hbm, table_hbm, out_hbm,
             idx_v, pe_v, buf0, buf1, buf2, buf3,
             gsem0, gsem1, gsem2, gsem3, wsem0, wsem1, wsem2, wsem3):
        g = lax.axis_index("c")
        l = lax.axis_index("s")
        s0 = g * sper
        pltpu.sync_copy(x_hbm.at[pl.ds(s0, sper), l], idx_v)
        pltpu.sync_copy(pe_hbm.at[pl.ds(s0 * d, sper * d)], pe_v)

        bufs = (buf0, buf1, buf2, buf3)
        gsems = (gsem0, gsem1, gsem2, gsem3)
        wsems = (wsem0, wsem1, wsem2, wsem3)

        def gather(m, bb, gsem):
            return pltpu.make_async_copy(
                table_hbm.at[idx_v.at[lax.div(m, nsub), lax.rem(m, nsub)]],
                bb, gsem)

        def write(m, bb, wsem):
            si = lax.div(m, nsub)
            sub = lax.rem(m, nsub)
            return pltpu.make_async_copy(
                bb,
                out_hbm.at[pl.ds((s0 + si) * b + l * stripe + sub * sub_tok,
                                 sub_tok), :],
                wsem)

        for p in range(2):
            gather(p, bufs[p], gsems[p]).start()

        @pl.loop(0, nchunk // 4)
        def _outer(t):
            for p in range(4):
                m = 4 * t + p
                si = lax.div(m, nsub)
                gather(m, bufs[p], gsems[p]).wait()

                peg = [pe_v[pl.ds(si * d + fb * _LANES, _LANES)]
                       for fb in range(groups)]

                @pl.loop(0, sub_tok, step=2)
                def _token(j0):
                    vecs = [bufs[p][j, pl.ds(fb * _LANES, _LANES)]
                            for j in (j0, j0 + 1) for fb in range(groups)]
                    scaled = [vv * scale for vv in vecs]
                    added = [scaled[tj * groups + fb] + peg[fb]
                             for tj in range(2) for fb in range(groups)]
                    for tj in (0, 1):
                        for fb in range(groups):
                            bufs[p][j0 + tj, pl.ds(fb * _LANES, _LANES)] = (
                                added[tj * groups + fb])

                write(m, bufs[p], wsems[p]).start()

                pn = (p + 2) % 4
                @pl.when(m + 2 < nchunk)
                def _next_gather():
                    @pl.when(m >= 2)
                    def _drain():
                        write(m - 2, bufs[pn], wsems[pn]).wait()
                    gather(m + 2, bufs[pn], gsems[pn]).start()

        for p in range(4):
            write(nchunk - 4 + p, bufs[p], wsems[p]).wait()

    out_phys = pl.kernel(
        body,
        out_type=jax.ShapeDtypeStruct((s * b, d), jnp.float32),
        mesh=mesh,
        compiler_params=pltpu.CompilerParams(use_tc_tiling_on_sc=False,
                                             needs_layout_passes=False),
        scratch_types=[
            pltpu.VMEM((sper, nsub, sub_tok), jnp.int32),
            pltpu.VMEM((sper * d,), jnp.float32),
            pltpu.VMEM((sub_tok, d), jnp.float32),
            pltpu.VMEM((sub_tok, d), jnp.float32),
            pltpu.VMEM((sub_tok, d), jnp.float32),
            pltpu.VMEM((sub_tok, d), jnp.float32),
            pltpu.SemaphoreType.DMA,
            pltpu.SemaphoreType.DMA,
            pltpu.SemaphoreType.DMA,
            pltpu.SemaphoreType.DMA,
            pltpu.SemaphoreType.DMA,
            pltpu.SemaphoreType.DMA,
            pltpu.SemaphoreType.DMA,
            pltpu.SemaphoreType.DMA,
        ],
    )(xtr, pe_flat, table)
    return out_phys.reshape(s, b, d).transpose(1, 0, 2)


# speedup vs baseline: 1.3354x; 1.1142x over previous
import math

import jax
import jax.numpy as jnp
from jax import lax
from jax.experimental import pallas as pl
from jax.experimental.pallas import tpu as pltpu
from jax.experimental.pallas import tpu_sc as plsc

_LANES = 16


def _positional_encoding_2d(seq_len, d):
    pos = jnp.arange(seq_len, dtype=jnp.float32)[:, None]
    even_idx = jnp.arange(0, d, 2, dtype=jnp.float32)
    odd_idx = jnp.arange(1, d, 2, dtype=jnp.float32)
    even_div = jnp.power(10000.0, 2.0 * even_idx / d)
    odd_div = jnp.power(10000.0, 2.0 * odd_idx / d)
    pe = jnp.zeros((seq_len, d), dtype=jnp.float32)
    pe = pe.at[:, 0::2].set(jnp.sin(pos / even_div))
    pe = pe.at[:, 1::2].set(jnp.cos(pos / odd_div))
    return pe


def kernel(x, table):
    b, s = x.shape
    v, d = table.shape
    scale = math.sqrt(d)

    info = plsc.get_sparse_core_info()
    nc, ns = info.num_cores, info.num_subcores

    sub_tok = 128
    nsub = 2
    stripe = nsub * sub_tok
    pitch = stripe + 1
    sper = s // nc
    assert b == ns * stripe and s % nc == 0 and d % _LANES == 0
    assert sper % 2 == 0
    groups = d // _LANES
    nrow = 8

    pe_flat = _positional_encoding_2d(s, d).reshape(-1)
    xtr = x.astype(jnp.int32).T.reshape(s, ns, nsub, sub_tok)

    mesh = plsc.VectorSubcoreMesh(core_axis_name="c", subcore_axis_name="s")

    def body(x_hbm, pe_hbm, table_hbm, out_hbm,
             idx_v, pe_v, gbuf0, gbuf1, wbig0, wbig1,
             gsem0, gsem1, wsem0, wsem1):
        g = lax.axis_index("c")
        l = lax.axis_index("s")
        lane = lax.iota(jnp.int32, _LANES)
        lanefb = [lane + fb * _LANES for fb in range(groups)]
        s0 = g * sper
        pltpu.sync_copy(x_hbm.at[pl.ds(s0, sper), l], idx_v)
        pltpu.sync_copy(pe_hbm.at[pl.ds(s0 * d, sper * d)], pe_v)

        def gather(si, sub, gbuf, gsem):
            return pltpu.make_async_copy(
                table_hbm.at[idx_v.at[si, sub]], gbuf, gsem)

        def row_dma(si, dt, wbig, wsem):
            return pltpu.make_async_copy(
                wbig.at[pl.ds(dt * nrow, nrow), pl.ds(0, stripe)],
                out_hbm.at[s0 + si, pl.ds(dt * nrow, nrow),
                           pl.ds(l * stripe, stripe)],
                wsem)

        gather(0, 0, gbuf0, gsem0).start()
        gather(0, 1, gbuf1, gsem1).start()

        @pl.loop(0, sper // 2)
        def _outer(t):
            for pos_par, sub, gbuf, gsem, wbig, wsem in (
                (0, 0, gbuf0, gsem0, wbig0, wsem0),
                (0, 1, gbuf1, gsem1, wbig0, wsem0),
                (1, 0, gbuf0, gsem0, wbig1, wsem1),
                (1, 1, gbuf1, gsem1, wbig1, wsem1),
            ):
                si = 2 * t + pos_par
                gather(si, sub, gbuf, gsem).wait()

                if sub == 0:
                    @pl.when(t > 0)
                    def _drain_prev_writes():
                        for dt in range(d // nrow):
                            row_dma(si - 2, dt, wbig, wsem).wait()

                peg = [pe_v[pl.ds(si * d + fb * _LANES, _LANES)]
                       for fb in range(groups)]

                @pl.loop(0, sub_tok, step=2)
                def _token(j0):
                    cols = [jnp.full((_LANES,), sub * sub_tok, jnp.int32) + j
                            for j in (j0, j0 + 1)]
                    vecs = [gbuf[j, pl.ds(fb * _LANES, _LANES)]
                            for j in (j0, j0 + 1) for fb in range(groups)]
                    scaled = [vv * scale for vv in vecs]
                    added = [scaled[tj * groups + fb] + peg[fb]
                             for tj in range(2) for fb in range(groups)]
                    for tj in range(2):
                        for fb in range(groups):
                            plsc.store_scatter(
                                wbig, [lanefb[fb], cols[tj]],
                                added[tj * groups + fb])

                if sub == nsub - 1:
                    for dt in range(d // nrow):
                        row_dma(si, dt, wbig, wsem).start()

                nxt = 2 * si + sub + 2
                @pl.when(nxt < 2 * sper)
                def _next_gather():
                    gather(lax.div(nxt, 2), lax.rem(nxt, 2),
                           gbuf, gsem).start()

        for dt in range(d // nrow):
            row_dma(sper - 2, dt, wbig0, wsem0).wait()
        for dt in range(d // nrow):
            row_dma(sper - 1, dt, wbig1, wsem1).wait()

    out_phys = pl.kernel(
        body,
        out_type=jax.ShapeDtypeStruct((s, d, b), jnp.float32),
        mesh=mesh,
        compiler_params=pltpu.CompilerParams(use_tc_tiling_on_sc=False,
                                             needs_layout_passes=False),
        scratch_types=[
            pltpu.VMEM((sper, nsub, sub_tok), jnp.int32),
            pltpu.VMEM((sper * d,), jnp.float32),
            pltpu.VMEM((sub_tok, d), jnp.float32),
            pltpu.VMEM((sub_tok, d), jnp.float32),
            pltpu.VMEM((d, pitch), jnp.float32),
            pltpu.VMEM((d, pitch), jnp.float32),
            pltpu.SemaphoreType.DMA,
            pltpu.SemaphoreType.DMA,
            pltpu.SemaphoreType.DMA,
            pltpu.SemaphoreType.DMA,
        ],
    )(xtr, pe_flat, table)
    return out_phys.transpose(2, 0, 1)
